# Initial kernel scaffold; baseline (speedup 1.0000x reference)
import jax
import jax.numpy as jnp
from jax.experimental import pallas as pl


def kernel(h0, label, x, edges, edge_attr, n_nodes, W_emb, b_emb, We1_0, be1_0, We2_0, be2_0, Wn1_0, bn1_0, Wn2_0, bn2_0, We1_1, be1_1, We2_1, be2_1, Wn1_1, bn1_1, Wn2_1, bn2_1, W_mu, b_mu, W_var, b_var):
    raise NotImplementedError("write your pallas kernel here")


# speedup vs baseline: 2.2313x; 2.2313x over previous
import functools

import jax
import jax.numpy as jnp
from jax import lax
from jax.experimental import pallas as pl
from jax.experimental.pallas import tpu as pltpu
from jax.experimental.pallas import tpu_sc as plsc

F32 = jnp.float32

NC = 2
NS = 16
NW = NC * NS
CH = 128

_MESH = dict(core_axis_name="c", subcore_axis_name="s", num_cores=NC,
             num_subcores=NS)


def _wid():
    return lax.axis_index("s") * NC + lax.axis_index("c")


@functools.lru_cache(maxsize=None)
def _sc_radial(n, ep, chunks):
    pt = chunks * CH

    @functools.partial(
        pl.kernel,
        out_type=jax.ShapeDtypeStruct((ep,), F32),
        mesh=plsc.VectorSubcoreMesh(**_MESH),
        compiler_params=pltpu.CompilerParams(needs_layout_passes=False),
        scratch_types=[
            pltpu.VMEM((n,), F32),
            pltpu.VMEM((n,), F32),
            pltpu.VMEM((n,), F32),
            pltpu.VMEM((chunks, CH), jnp.int32),
            pltpu.VMEM((chunks, CH), jnp.int32),
            pltpu.VMEM((pt,), F32),
        ],
    )
    def k(x0_hbm, x1_hbm, x2_hbm, row2_hbm, col2_hbm, rad_hbm,
          x0v, x1v, x2v, idxr, idxc, radv):
        w = _wid()
        pltpu.sync_copy(x0_hbm, x0v)
        pltpu.sync_copy(x1_hbm, x1v)
        pltpu.sync_copy(x2_hbm, x2v)
        pltpu.sync_copy(row2_hbm.at[pl.ds(w * chunks, chunks)], idxr)
        pltpu.sync_copy(col2_hbm.at[pl.ds(w * chunks, chunks)], idxc)

        def chunk(j, _):
            for g in range(CH // 16):
                rv = idxr[j, pl.ds(g * 16, 16)]
                cv = idxc[j, pl.ds(g * 16, 16)]
                acc = jnp.zeros((16,), F32)
                for xv in (x0v, x1v, x2v):
                    diff = (plsc.load_gather(xv, [rv])
                            - plsc.load_gather(xv, [cv]))
                    acc = acc + diff * diff
                radv[pl.ds(j * CH + g * 16, 16)] = acc
            return 0

        lax.fori_loop(0, chunks, chunk, 0)
        pltpu.sync_copy(radv, rad_hbm.at[pl.ds(w * pt, pt)])

    return k


@functools.lru_cache(maxsize=None)
def _sc_gather(n, hid, ep, chunks):
    @functools.partial(
        pl.kernel,
        out_type=jax.ShapeDtypeStruct((ep, hid), F32),
        mesh=plsc.VectorSubcoreMesh(**_MESH),
        scratch_types=[
            pltpu.VMEM((chunks, CH), jnp.int32),
            pltpu.VMEM((chunks, CH), jnp.int32),
            pltpu.VMEM((CH, hid), F32),
            pltpu.VMEM((CH, hid), F32),
            pltpu.SemaphoreType.DMA,
            pltpu.SemaphoreType.DMA,
        ],
    )
    def k(ha_hbm, hb_hbm, row2_hbm, col2_hbm, g_hbm,
          idxr, idxc, bufa, bufb, sema, semb):
        w = _wid()
        pltpu.sync_copy(row2_hbm.at[pl.ds(w * chunks, chunks)], idxr)
        pltpu.sync_copy(col2_hbm.at[pl.ds(w * chunks, chunks)], idxc)

        def chunk(j, _):
            a = pltpu.async_copy(ha_hbm.at[idxr.at[j]], bufa, sema)
            b = pltpu.async_copy(hb_hbm.at[idxc.at[j]], bufb, semb)
            a.wait()
            b.wait()

            def addrow(r, _):
                for g in range(hid // 16):
                    sl = pl.ds(g * 16, 16)
                    bufa[r, sl] = bufa[r, sl] + bufb[r, sl]
                return 0

            lax.fori_loop(0, CH, addrow, 0)
            pltpu.sync_copy(bufa, g_hbm.at[pl.ds((w * chunks + j) * CH, CH)])
            return 0

        lax.fori_loop(0, chunks, chunk, 0)

    return k


@functools.lru_cache(maxsize=None)
def _sc_scatter(np_, hid, ep, chunks):
    rps = np_ // NS
    assert rps % CH == 0

    @functools.partial(
        pl.kernel,
        out_type=jax.ShapeDtypeStruct((NC, np_, hid), F32),
        mesh=plsc.VectorSubcoreMesh(**_MESH),
        scratch_types=[
            pltpu.VMEM((chunks, CH), jnp.int32),
            pltpu.VMEM((CH, hid), F32),
            pltpu.VMEM_SHARED((np_, hid), F32),
        ],
    )
    def k(m2_hbm, rows2_hbm, zero_hbm, parts_hbm, idx, buf, shared):
        c = lax.axis_index("c")
        s = lax.axis_index("s")
        w = s * NC + c
        for t in range(rps // CH):
            pltpu.sync_copy(zero_hbm,
                            shared.at[pl.ds(s * rps + t * CH, CH)])
        plsc.subcore_barrier()

        pltpu.sync_copy(rows2_hbm.at[pl.ds(w * chunks, chunks)], idx)

        def chunk(j, _):
            pltpu.sync_copy(m2_hbm.at[pl.ds((w * chunks + j) * CH, CH)], buf)
            pltpu.sync_copy(buf, shared.at[idx.at[j]], add=True)
            return 0

        lax.fori_loop(0, chunks, chunk, 0)
        plsc.subcore_barrier()

        for t in range(rps // CH):
            base = s * rps + t * CH
            pltpu.sync_copy(shared.at[pl.ds(base, CH)], buf)
            pltpu.sync_copy(buf, parts_hbm.at[c, pl.ds(base, CH)])

    return k


def _dot(a, b):
    return jnp.dot(a, b, preferred_element_type=F32)


def _elu(x):
    return jnp.where(x > 0, x, jnp.exp(x) - 1.0)


def _tc_pre(h0, w_emb, b_emb, a0, b0, nb):
    n = h0.shape[0]

    def body(h0_r, we_r, be_r, a_r, b_r, h_r, ha_r, hb_r):
        h = _dot(h0_r[...], we_r[...]) + be_r[...]
        h_r[...] = h
        ha_r[...] = _dot(h, a_r[...])
        hb_r[...] = _dot(h, b_r[...])

    full = lambda s: pl.BlockSpec(s, lambda i: (0, 0))
    return pl.pallas_call(
        body,
        grid=(n // nb,),
        in_specs=[
            pl.BlockSpec((nb, h0.shape[1]), lambda i: (i, 0)),
            full(w_emb.shape), full(b_emb.shape), full(a0.shape),
            full(b0.shape),
        ],
        out_specs=[pl.BlockSpec((nb, 128), lambda i: (i, 0))] * 3,
        out_shape=[jax.ShapeDtypeStruct((n, 128), F32)] * 3,
    )(h0, w_emb, b_emb, a0, b0)


def _tc_edge(g, radial, ea, wr, wea, be1, we2, be2, be):
    ep, hid = g.shape

    def body(g_r, rad_r, ea_r, wr_r, wea_r, be1_r, we2_r, be2_r, m2_r):
        pre = (g_r[...] + rad_r[...] * wr_r[...]
               + _dot(ea_r[...], wea_r[...]) + be1_r[...])
        m = _elu(pre)
        m2_r[...] = _elu(_dot(m, we2_r[...]) + be2_r[...])

    full = lambda s: pl.BlockSpec(s, lambda i: (0, 0))
    return pl.pallas_call(
        body,
        grid=(ep // be,),
        in_specs=[
            pl.BlockSpec((be, hid), lambda i: (i, 0)),
            pl.BlockSpec((be, 1), lambda i: (i, 0)),
            pl.BlockSpec((be, ea.shape[1]), lambda i: (i, 0)),
            full(wr.shape), full(wea.shape), full(be1.shape),
            full(we2.shape), full(be2.shape),
        ],
        out_specs=pl.BlockSpec((be, hid), lambda i: (i, 0)),
        out_shape=jax.ShapeDtypeStruct((ep, hid), F32),
    )(g, radial, ea, wr, wea, be1, we2, be2)


def _tc_node_mid(h, parts, wh, wa, bn1, wn2, bn2, a1, b1, nb):
    n, hid = h.shape

    def body(h_r, p_r, wh_r, wa_r, bn1_r, wn2_r, bn2_r, a_r, b_r,
             h_out, ha_out, hb_out):
        agg = p_r[0] + p_r[1]
        t = _elu(_dot(h_r[...], wh_r[...]) + _dot(agg, wa_r[...]) + bn1_r[...])
        hn = _dot(t, wn2_r[...]) + bn2_r[...]
        h_out[...] = hn
        ha_out[...] = _dot(hn, a_r[...])
        hb_out[...] = _dot(hn, b_r[...])

    full = lambda s: pl.BlockSpec(s, lambda i: (0, 0))
    return pl.pallas_call(
        body,
        grid=(n // nb,),
        in_specs=[
            pl.BlockSpec((nb, hid), lambda i: (i, 0)),
            pl.BlockSpec((NC, nb, hid), lambda i: (0, i, 0)),
            full(wh.shape), full(wa.shape), full(bn1.shape),
            full(wn2.shape), full(bn2.shape), full(a1.shape), full(b1.shape),
        ],
        out_specs=[pl.BlockSpec((nb, hid), lambda i: (i, 0))] * 3,
        out_shape=[jax.ShapeDtypeStruct((n, hid), F32)] * 3,
    )(h, parts, wh, wa, bn1, wn2, bn2, a1, b1)


def _tc_node_final(h, parts, label, eps, wh, wa, bn1, wn2, bn2,
                   wmu_h, wmu_l, bmu, wv_h, wv_l, bv, nb):
    n, hid = h.shape
    lat = wmu_h.shape[1]

    def body(h_r, p_r, lab_r, eps_r, wh_r, wa_r, bn1_r, wn2_r, bn2_r,
             wmh_r, wml_r, bmu_r, wvh_r, wvl_r, bv_r, z_r):
        agg = p_r[0] + p_r[1]
        t = _elu(_dot(h_r[...], wh_r[...]) + _dot(agg, wa_r[...]) + bn1_r[...])
        hn = _dot(t, wn2_r[...]) + bn2_r[...]
        mu = _dot(hn, wmh_r[...]) + _dot(lab_r[...], wml_r[...]) + bmu_r[...]
        lv = _dot(hn, wvh_r[...]) + _dot(lab_r[...], wvl_r[...]) + bv_r[...]
        z_r[...] = mu + 0.01 * eps_r[...] * jnp.exp(lv * 0.5)

    full = lambda s: pl.BlockSpec(s, lambda i: (0, 0))
    return pl.pallas_call(
        body,
        grid=(n // nb,),
        in_specs=[
            pl.BlockSpec((nb, hid), lambda i: (i, 0)),
            pl.BlockSpec((NC, nb, hid), lambda i: (0, i, 0)),
            pl.BlockSpec((nb, label.shape[1]), lambda i: (i, 0)),
            pl.BlockSpec((nb, lat), lambda i: (i, 0)),
            full(wh.shape), full(wa.shape), full(bn1.shape),
            full(wn2.shape), full(bn2.shape), full(wmu_h.shape),
            full(wmu_l.shape), full(bmu.shape), full(wv_h.shape),
            full(wv_l.shape), full(bv.shape),
        ],
        out_specs=pl.BlockSpec((nb, lat), lambda i: (i, 0)),
        out_shape=jax.ShapeDtypeStruct((n, lat), F32),
    )(h, parts, label, eps, wh, wa, bn1, wn2, bn2,
      wmu_h, wmu_l, bmu, wv_h, wv_l, bv)


def kernel(h0, label, x, edges, edge_attr, n_nodes, W_emb, b_emb,
           We1_0, be1_0, We2_0, be2_0, Wn1_0, bn1_0, Wn2_0, bn2_0,
           We1_1, be1_1, We2_1, be2_1, Wn1_1, bn1_1, Wn2_1, bn2_1,
           W_mu, b_mu, W_var, b_var):
    n, hid = h0.shape[0], W_emb.shape[1]
    e = edges.shape[1]
    lat = W_mu.shape[1]
    nb = 2000

    chunks = -(-(-(-e // (NW * CH))) // 8) * 8
    ep = NW * CH * chunks
    pad = ep - e
    np_ = -(-(n + 1) // (NS * CH)) * (NS * CH)

    row, col = edges[0], edges[1]
    row_g = jnp.concatenate([row, jnp.zeros((pad,), jnp.int32)])
    col_g = jnp.concatenate([col, jnp.zeros((pad,), jnp.int32)])
    row_s = jnp.concatenate([row, jnp.full((pad,), n, jnp.int32)])
    row2 = row_g.reshape(ep // CH, CH)
    col2 = col_g.reshape(ep // CH, CH)
    rows2 = row_s.reshape(ep // CH, CH)
    ea_p = jnp.concatenate(
        [edge_attr, jnp.zeros((pad, edge_attr.shape[1]), F32)], axis=0)
    zero_blk = jnp.zeros((CH, hid), F32)
    eps = jax.random.normal(jax.random.key(1), (n, lat), dtype=F32)

    def esplit(we1):
        return (we1[:hid], we1[hid:2 * hid], we1[2 * hid:2 * hid + 1],
                we1[2 * hid + 1:])

    a0, b0, wr0, wea0 = esplit(We1_0)
    a1, b1, wr1, wea1 = esplit(We1_1)
    wh0, wa0 = Wn1_0[:hid] + Wn1_0[2 * hid:], Wn1_0[hid:2 * hid]
    wh1, wa1 = Wn1_1[:hid] + Wn1_1[2 * hid:], Wn1_1[hid:2 * hid]
    r2 = lambda v: v.reshape(1, -1)

    h, ha, hb = _tc_pre(h0, W_emb, r2(b_emb), a0, b0, nb)
    radial = _sc_radial(n, ep, chunks)(
        x[:, 0], x[:, 1], x[:, 2], row2, col2).reshape(ep, 1)

    g = _sc_gather(n, hid, ep, chunks)(ha, hb, row2, col2)
    m2 = _tc_edge(g, radial, ea_p, wr0, wea0, r2(be1_0), We2_0, r2(be2_0),
                  2048)
    parts = _sc_scatter(np_, hid, ep, chunks)(m2, rows2, zero_blk)
    h, ha, hb = _tc_node_mid(h, parts, wh0, wa0, r2(bn1_0), Wn2_0, r2(bn2_0),
                             a1, b1, nb)

    g = _sc_gather(n, hid, ep, chunks)(ha, hb, row2, col2)
    m2 = _tc_edge(g, radial, ea_p, wr1, wea1, r2(be1_1), We2_1, r2(be2_1),
                  2048)
    parts = _sc_scatter(np_, hid, ep, chunks)(m2, rows2, zero_blk)
    z = _tc_node_final(h, parts, label, eps, wh1, wa1, r2(bn1_1), Wn2_1,
                       r2(bn2_1), W_mu[:hid], W_mu[hid:], r2(b_mu),
                       W_var[:hid], W_var[hid:], r2(b_var), nb)
    return z
